# speedup vs baseline: 1.6107x; 1.6107x over previous
import functools
import math

import numpy as np
import jax
import jax.numpy as jnp
from jax.experimental import pallas as pl
from jax.experimental.pallas import tpu as pltpu

_SR = 22050
_F_MIN = 32.7
_BPO = 12
_HOP = 256


@functools.lru_cache(maxsize=None)
def _cqt_constants():
    f_max = _SR / 2.0
    q = 1.0 / (2.0 ** (1.0 / _BPO) - 1.0)
    n_bins = math.ceil(_BPO * math.log2(f_max / _F_MIN))
    fft_len = 1 << (int(math.ceil(q * _SR / _F_MIN)) - 1).bit_length()

    temporal = np.zeros((n_bins, fft_len), dtype=np.complex128)
    for k in range(n_bins):
        f_k = _F_MIN * 2.0 ** (k / _BPO)
        n_k = 2 * round(q * _SR / f_k / 2) + 1
        n = np.arange(-(n_k - 1) // 2, (n_k - 1) // 2 + 1)
        w = np.hamming(n_k) / n_k
        start = fft_len // 2 + n[0]
        temporal[k, start:start + n_k] = w * np.exp(2j * np.pi * q / n_k * n)
    spectral = np.fft.fft(temporal, axis=-1) / fft_len
    folded = np.fft.fft(spectral, axis=-1).T

    c_int = np.zeros((fft_len, 2 * n_bins), dtype=np.float64)
    c_int[:, 0::2] = folded.real
    c_int[:, 1::2] = folded.imag

    row_amp = np.abs(c_int).max(axis=1)
    nz = np.nonzero(row_amp > row_amp.max() * 1e-7)[0]
    s0 = int(nz[0]) // _HOP
    s1 = int(nz[-1]) // _HOP + 1
    ns = s1 - s0

    kp = 256
    c_band = np.zeros((ns * _HOP, kp), np.float32)
    c_band[:, :2 * n_bins] = c_int[s0 * _HOP:s1 * _HOP]
    return {
        "n_bins": n_bins,
        "fft_len": fft_len,
        "s0": s0,
        "ns": ns,
        "c_band": jnp.asarray(c_band, jnp.bfloat16),
    }


def _cqt_pallas(y, c_band, *, n_frames, ns, kout):
    batch, rows, hop = y.shape
    ns_hop = ns * hop

    def body(y_ref, c_ref, o_ref, frm_ref):
        for s in range(ns):
            frm_ref[:, s * hop:(s + 1) * hop] = y_ref[0, s:s + n_frames, :]
        o_ref[0] = jnp.dot(frm_ref[...], c_ref[...],
                           preferred_element_type=jnp.float32)[:, :kout]

    return pl.pallas_call(
        body,
        out_shape=jax.ShapeDtypeStruct((batch, n_frames, kout), jnp.float32),
        grid=(batch,),
        in_specs=[
            pl.BlockSpec((1, rows, hop), lambda b: (b, 0, 0)),
            pl.BlockSpec((ns_hop, c_band.shape[1]), lambda b: (0, 0)),
        ],
        out_specs=pl.BlockSpec((1, n_frames, kout), lambda b: (b, 0, 0)),
        scratch_shapes=[pltpu.VMEM((n_frames, ns_hop), jnp.bfloat16)],
        compiler_params=pltpu.CompilerParams(
            dimension_semantics=("parallel",)),
    )(y, c_band)


def kernel(x):
    cst = _cqt_constants()
    n_bins, fft_len = cst["n_bins"], cst["fft_len"]
    s0, ns = cst["s0"], cst["ns"]

    x = jnp.asarray(x, jnp.float32)
    lead, t_len = x.shape[:-1], x.shape[-1]
    x2 = x.reshape(-1, t_len)
    batch = x2.shape[0]
    n_frames = (t_len - 1) // _HOP + 1

    rows = -(-(n_frames - 1 + ns) // 8) * 8
    lpad = fft_len // 2 - s0 * _HOP
    rpad = rows * _HOP - lpad - t_len
    y = jnp.pad(x2.astype(jnp.bfloat16), ((0, 0), (lpad, rpad)))
    y = y.reshape(batch, rows, _HOP)

    out = _cqt_pallas(y, cst["c_band"], n_frames=n_frames, ns=ns,
                      kout=2 * n_bins)
    return out.reshape(*lead, n_frames, n_bins, 2)


# speedup vs baseline: 1.6229x; 1.0076x over previous
import functools
import math

import numpy as np
import jax
import jax.numpy as jnp
from jax.experimental import pallas as pl
from jax.experimental.pallas import tpu as pltpu

_SR = 22050
_F_MIN = 32.7
_BPO = 12
_HOP = 256


@functools.lru_cache(maxsize=None)
def _cqt_constants():
    f_max = _SR / 2.0
    q = 1.0 / (2.0 ** (1.0 / _BPO) - 1.0)
    n_bins = math.ceil(_BPO * math.log2(f_max / _F_MIN))
    fft_len = 1 << (int(math.ceil(q * _SR / _F_MIN)) - 1).bit_length()

    temporal = np.zeros((n_bins, fft_len), dtype=np.complex128)
    for k in range(n_bins):
        f_k = _F_MIN * 2.0 ** (k / _BPO)
        n_k = 2 * round(q * _SR / f_k / 2) + 1
        n = np.arange(-(n_k - 1) // 2, (n_k - 1) // 2 + 1)
        w = np.hamming(n_k) / n_k
        start = fft_len // 2 + n[0]
        temporal[k, start:start + n_k] = w * np.exp(2j * np.pi * q / n_k * n)
    spectral = np.fft.fft(temporal, axis=-1) / fft_len
    folded = np.fft.fft(spectral, axis=-1).T

    c_int = np.zeros((fft_len, 2 * n_bins), dtype=np.float64)
    c_int[:, 0::2] = folded.real
    c_int[:, 1::2] = folded.imag

    row_amp = np.abs(c_int).max(axis=1)
    nz = np.nonzero(row_amp > row_amp.max() * 1e-7)[0]
    s0 = int(nz[0]) // _HOP
    s1 = int(nz[-1]) // _HOP + 1
    ns = s1 - s0

    vg = 16
    ug = -(-ns // vg)
    c3 = np.zeros((ug * vg * _HOP, 256), np.float32)
    for s in range(ns):
        u, v = divmod(s, vg)
        c3[(u * vg + v) * _HOP:(u * vg + v + 1) * _HOP, :2 * n_bins] = (
            c_int[(s0 + s) * _HOP:(s0 + s + 1) * _HOP])
    return {
        "n_bins": n_bins,
        "fft_len": fft_len,
        "s0": s0,
        "ns": ns,
        "vg": vg,
        "ug": ug,
        "c3": jnp.asarray(c3, jnp.bfloat16),
    }


def _cqt_pallas(x4, c3, *, n_frames, lead_zero, sig_rows, vg, ug, kout):
    batch = x4.shape[0]
    hop = x4.shape[2]
    kv = vg * hop
    zrows = n_frames + (ug - 1) * vg
    yrows = -(-(zrows + vg - 1) // vg) * vg

    def body(x_ref, c_ref, o_ref, ybuf, zbuf):
        ybuf[:lead_zero, :] = jnp.zeros((lead_zero, hop), jnp.bfloat16)
        ybuf[lead_zero:lead_zero + sig_rows, :] = x_ref[0].astype(jnp.bfloat16)
        ybuf[lead_zero + sig_rows:, :] = jnp.zeros(
            (yrows - lead_zero - sig_rows, hop), jnp.bfloat16)
        for v in range(vg):
            zbuf[:, v * hop:(v + 1) * hop] = ybuf[v:v + zrows, :]
        acc = jnp.dot(zbuf[0:n_frames, :], c_ref[0:kv, :],
                      preferred_element_type=jnp.float32)
        for u in range(1, ug):
            acc = acc + jnp.dot(zbuf[u * vg:u * vg + n_frames, :],
                                c_ref[u * kv:(u + 1) * kv, :],
                                preferred_element_type=jnp.float32)
        o_ref[0] = acc[:, :kout]

    return pl.pallas_call(
        body,
        out_shape=jax.ShapeDtypeStruct((batch, n_frames, kout), jnp.float32),
        grid=(batch,),
        in_specs=[
            pl.BlockSpec((1, sig_rows, hop), lambda b: (b, 0, 0)),
            pl.BlockSpec((ug * kv, c3.shape[1]), lambda b: (0, 0)),
        ],
        out_specs=pl.BlockSpec((1, n_frames, kout), lambda b: (b, 0, 0)),
        scratch_shapes=[
            pltpu.VMEM((yrows, hop), jnp.bfloat16),
            pltpu.VMEM((zrows, kv), jnp.bfloat16),
        ],
        compiler_params=pltpu.CompilerParams(
            dimension_semantics=("parallel",)),
    )(x4, c3)


def kernel(x):
    cst = _cqt_constants()
    n_bins, fft_len = cst["n_bins"], cst["fft_len"]
    s0 = cst["s0"]

    x = jnp.asarray(x, jnp.float32)
    lead, t_len = x.shape[:-1], x.shape[-1]
    x2 = x.reshape(-1, t_len)
    batch = x2.shape[0]
    n_frames = (t_len - 1) // _HOP + 1

    sig_rows = -(-t_len // _HOP)
    if t_len % _HOP:
        x2 = jnp.pad(x2, ((0, 0), (0, sig_rows * _HOP - t_len)))
    x4 = x2.reshape(batch, sig_rows, _HOP)

    lead_zero = fft_len // 2 // _HOP - s0
    out = _cqt_pallas(x4, cst["c3"], n_frames=n_frames, lead_zero=lead_zero,
                      sig_rows=sig_rows, vg=cst["vg"], ug=cst["ug"],
                      kout=2 * n_bins)
    return out.reshape(*lead, n_frames, n_bins, 2)


# speedup vs baseline: 2.2379x; 1.3789x over previous
import functools
import math

import numpy as np
import jax
import jax.numpy as jnp
from jax.experimental import pallas as pl
from jax.experimental.pallas import tpu as pltpu

_SR = 22050
_F_MIN = 32.7
_BPO = 12
_HOP = 256


@functools.lru_cache(maxsize=None)
def _cqt_constants():
    f_max = _SR / 2.0
    q = 1.0 / (2.0 ** (1.0 / _BPO) - 1.0)
    n_bins = math.ceil(_BPO * math.log2(f_max / _F_MIN))
    fft_len = 1 << (int(math.ceil(q * _SR / _F_MIN)) - 1).bit_length()

    temporal = np.zeros((n_bins, fft_len), dtype=np.complex128)
    for k in range(n_bins):
        f_k = _F_MIN * 2.0 ** (k / _BPO)
        n_k = 2 * round(q * _SR / f_k / 2) + 1
        n = np.arange(-(n_k - 1) // 2, (n_k - 1) // 2 + 1)
        w = np.hamming(n_k) / n_k
        start = fft_len // 2 + n[0]
        temporal[k, start:start + n_k] = w * np.exp(2j * np.pi * q / n_k * n)
    spectral = np.fft.fft(temporal, axis=-1) / fft_len
    folded = np.fft.fft(spectral, axis=-1).T

    c_int = np.zeros((fft_len, 2 * n_bins), dtype=np.float64)
    c_int[:, 0::2] = folded.real
    c_int[:, 1::2] = folded.imag

    row_amp = np.abs(c_int).max(axis=1)
    nz = np.nonzero(row_amp > row_amp.max() * 1e-7)[0]
    s0 = int(nz[0]) // _HOP
    s1 = int(nz[-1]) // _HOP + 1
    ns = s1 - s0

    c_pad = np.zeros((fft_len, 256), np.float64)
    c_pad[:, :2 * n_bins] = c_int
    hi_amp = np.abs(c_pad[:, 128:]).max(axis=1)
    hz = np.nonzero(hi_amp > row_amp.max() * 1e-7)[0]
    hr0 = int(hz[0]) // _HOP - s0
    hr1 = int(hz[-1]) // _HOP + 1 - s0

    sp = (ns + 1) // 2
    nv = sp
    c_d1 = np.zeros((sp * _HOP, 256), np.float32)
    for t in range(sp):
        c_d1[t * _HOP:(t + 1) * _HOP, :128] = c_pad[(s0 + t) * _HOP:
                                                    (s0 + t + 1) * _HOP, :128]
        if sp + t < ns:
            c_d1[t * _HOP:(t + 1) * _HOP, 128:] = c_pad[
                (s0 + sp + t) * _HOP:(s0 + sp + t + 1) * _HOP, :128]
    nh = hr1 - hr0
    c_d2 = np.zeros((nh * _HOP, 128), np.float32)
    for t in range(nh):
        c_d2[t * _HOP:(t + 1) * _HOP] = c_pad[(s0 + hr0 + t) * _HOP:
                                              (s0 + hr0 + t + 1) * _HOP, 128:]
    return {
        "n_bins": n_bins,
        "fft_len": fft_len,
        "s0": s0,
        "ns": ns,
        "sp": sp,
        "hr0": hr0,
        "nh": nh,
        "nv": nv,
        "c_d1": jnp.asarray(c_d1, jnp.bfloat16),
        "c_d2": jnp.asarray(c_d2, jnp.bfloat16),
    }


def _cqt_pallas(x4, c_d1, c_d2, *, n_frames, lead_zero, sig_rows, cst, kout):
    batch = x4.shape[0]
    hop = x4.shape[2]
    sp, hr0, nh, nv = cst["sp"], cst["hr0"], cst["nh"], cst["nv"]
    mp = -(-(n_frames + sp) // 8) * 8
    zrows = -(-mp // 16) * 16
    yrows = -(-(zrows + nv - 1) // 16) * 16

    def body(x_ref, c1_ref, c2_ref, o_ref, ybuf, zbuf):
        ybuf[:lead_zero, :] = jnp.zeros((lead_zero, hop), jnp.bfloat16)
        ybuf[lead_zero:lead_zero + sig_rows, :] = x_ref[0].astype(jnp.bfloat16)
        ybuf[lead_zero + sig_rows:, :] = jnp.zeros(
            (yrows - lead_zero - sig_rows, hop), jnp.bfloat16)
        for v in range(nv):
            zbuf[:, v * hop:(v + 1) * hop] = ybuf[v:v + zrows, :]
        d1 = jnp.dot(zbuf[0:mp, :], c1_ref[...],
                     preferred_element_type=jnp.float32)
        d2 = jnp.dot(zbuf[0:mp, 0:nh * hop], c2_ref[...],
                     preferred_element_type=jnp.float32)
        lo = d1[0:n_frames, 0:128] + d1[sp:sp + n_frames, 128:256]
        hi = d2[hr0:hr0 + n_frames, 0:kout - 128]
        o_ref[0] = jnp.concatenate([lo, hi], axis=1)

    return pl.pallas_call(
        body,
        out_shape=jax.ShapeDtypeStruct((batch, n_frames, kout), jnp.float32),
        grid=(batch,),
        in_specs=[
            pl.BlockSpec((1, sig_rows, hop), lambda b: (b, 0, 0)),
            pl.BlockSpec(c_d1.shape, lambda b: (0, 0)),
            pl.BlockSpec(c_d2.shape, lambda b: (0, 0)),
        ],
        out_specs=pl.BlockSpec((1, n_frames, kout), lambda b: (b, 0, 0)),
        scratch_shapes=[
            pltpu.VMEM((yrows, hop), jnp.bfloat16),
            pltpu.VMEM((zrows, nv * hop), jnp.bfloat16),
        ],
        compiler_params=pltpu.CompilerParams(
            dimension_semantics=("parallel",)),
    )(x4, c_d1, c_d2)


def kernel(x):
    cst = _cqt_constants()
    n_bins, fft_len = cst["n_bins"], cst["fft_len"]
    s0 = cst["s0"]

    x = jnp.asarray(x, jnp.float32)
    lead, t_len = x.shape[:-1], x.shape[-1]
    x2 = x.reshape(-1, t_len)
    batch = x2.shape[0]
    n_frames = (t_len - 1) // _HOP + 1

    sig_rows = -(-t_len // _HOP)
    if t_len % _HOP:
        x2 = jnp.pad(x2, ((0, 0), (0, sig_rows * _HOP - t_len)))
    x4 = x2.reshape(batch, sig_rows, _HOP)

    lead_zero = fft_len // 2 // _HOP - s0
    out = _cqt_pallas(x4, cst["c_d1"], cst["c_d2"], n_frames=n_frames,
                      lead_zero=lead_zero, sig_rows=sig_rows, cst=cst,
                      kout=2 * n_bins)
    return out.reshape(*lead, n_frames, n_bins, 2)


# speedup vs baseline: 2.4882x; 1.1119x over previous
import functools
import math

import numpy as np
import jax
import jax.numpy as jnp
from jax.experimental import pallas as pl
from jax.experimental.pallas import tpu as pltpu

_SR = 22050
_F_MIN = 32.7
_BPO = 12
_HOP = 256


@functools.lru_cache(maxsize=None)
def _cqt_constants():
    f_max = _SR / 2.0
    q = 1.0 / (2.0 ** (1.0 / _BPO) - 1.0)
    n_bins = math.ceil(_BPO * math.log2(f_max / _F_MIN))
    fft_len = 1 << (int(math.ceil(q * _SR / _F_MIN)) - 1).bit_length()

    temporal = np.zeros((n_bins, fft_len), dtype=np.complex128)
    for k in range(n_bins):
        f_k = _F_MIN * 2.0 ** (k / _BPO)
        n_k = 2 * round(q * _SR / f_k / 2) + 1
        n = np.arange(-(n_k - 1) // 2, (n_k - 1) // 2 + 1)
        w = np.hamming(n_k) / n_k
        start = fft_len // 2 + n[0]
        temporal[k, start:start + n_k] = w * np.exp(2j * np.pi * q / n_k * n)
    spectral = np.fft.fft(temporal, axis=-1) / fft_len
    folded = np.fft.fft(spectral, axis=-1).T

    c_int = np.zeros((fft_len, 2 * n_bins), dtype=np.float64)
    c_int[:, 0::2] = folded.real
    c_int[:, 1::2] = folded.imag

    row_amp = np.abs(c_int).max(axis=1)
    nz = np.nonzero(row_amp > row_amp.max() * 1e-7)[0]
    s0 = int(nz[0]) // _HOP
    s1 = int(nz[-1]) // _HOP + 1
    ns = s1 - s0

    c_pad = np.zeros((fft_len, 256), np.float64)
    c_pad[:, :2 * n_bins] = c_int
    hi_amp = np.abs(c_pad[:, 128:]).max(axis=1)
    hz = np.nonzero(hi_amp > row_amp.max() * 1e-7)[0]
    hr0 = int(hz[0]) // _HOP - s0
    hr1 = int(hz[-1]) // _HOP + 1 - s0

    def _band(cols):
        amp = np.abs(c_pad[:, cols]).max(axis=1)
        nzc = np.nonzero(amp > row_amp.max() * 1e-7)[0]
        return int(nzc[0]) // _HOP - s0, int(nzc[-1]) // _HOP + 1 - s0

    aS, aE = _band(slice(0, 64))
    bS, bE = _band(slice(64, 128))
    k1 = max(-(-(aE - aS) // 3), bE - bS)
    nv = k1
    c_d1 = np.zeros((k1 * _HOP, 256), np.float32)
    for t in range(k1):
        for m in range(3):
            s = aS + m * k1 + t
            if s < ns:
                c_d1[t * _HOP:(t + 1) * _HOP, 64 * m:64 * (m + 1)] = (
                    c_pad[(s0 + s) * _HOP:(s0 + s + 1) * _HOP, :64])
        s = bS + t
        if s < bE:
            c_d1[t * _HOP:(t + 1) * _HOP, 192:256] = (
                c_pad[(s0 + s) * _HOP:(s0 + s + 1) * _HOP, 64:128])
    nh = hr1 - hr0
    c_d2 = np.zeros((nh * _HOP, 128), np.float32)
    for t in range(nh):
        c_d2[t * _HOP:(t + 1) * _HOP] = c_pad[(s0 + hr0 + t) * _HOP:
                                              (s0 + hr0 + t + 1) * _HOP, 128:]
    return {
        "n_bins": n_bins,
        "fft_len": fft_len,
        "s0": s0,
        "ns": ns,
        "k1": k1,
        "aS": aS,
        "bS": bS,
        "hr0": hr0,
        "nh": nh,
        "nv": nv,
        "c_d1": jnp.asarray(c_d1, jnp.bfloat16),
        "c_d2": jnp.asarray(c_d2, jnp.bfloat16),
    }


def _cqt_pallas(x4, c_d1, c_d2, *, n_frames, lead_zero, sig_rows, cst, kout):
    batch = x4.shape[0]
    hop = x4.shape[2]
    k1, aS, bS = cst["k1"], cst["aS"], cst["bS"]
    hr0, nh, nv = cst["hr0"], cst["nh"], cst["nv"]
    off_max = max(aS + 2 * k1, bS, hr0)
    mp = -(-(n_frames + off_max) // 8) * 8
    zrows = -(-mp // 16) * 16
    yrows = -(-(zrows + nv - 1) // 16) * 16

    def body(x_ref, c1_ref, c2_ref, o_ref, ybuf, zbuf):
        ybuf[:lead_zero, :] = jnp.zeros((lead_zero, hop), jnp.bfloat16)
        ybuf[lead_zero:lead_zero + sig_rows, :] = x_ref[0].astype(jnp.bfloat16)
        ybuf[lead_zero + sig_rows:, :] = jnp.zeros(
            (yrows - lead_zero - sig_rows, hop), jnp.bfloat16)
        for v in range(nv):
            zbuf[:, v * hop:(v + 1) * hop] = ybuf[v:v + zrows, :]
        d1 = jnp.dot(zbuf[0:mp, :], c1_ref[...],
                     preferred_element_type=jnp.float32)
        d2 = jnp.dot(zbuf[0:mp, 0:nh * hop], c2_ref[...],
                     preferred_element_type=jnp.float32)
        lo_a = (d1[aS:aS + n_frames, 0:64]
                + d1[aS + k1:aS + k1 + n_frames, 64:128]
                + d1[aS + 2 * k1:aS + 2 * k1 + n_frames, 128:192])
        lo_b = d1[bS:bS + n_frames, 192:256]
        hi = d2[hr0:hr0 + n_frames, 0:kout - 128]
        o_ref[0] = jnp.concatenate([lo_a, lo_b, hi], axis=1)

    return pl.pallas_call(
        body,
        out_shape=jax.ShapeDtypeStruct((batch, n_frames, kout), jnp.float32),
        grid=(batch,),
        in_specs=[
            pl.BlockSpec((1, sig_rows, hop), lambda b: (b, 0, 0)),
            pl.BlockSpec(c_d1.shape, lambda b: (0, 0)),
            pl.BlockSpec(c_d2.shape, lambda b: (0, 0)),
        ],
        out_specs=pl.BlockSpec((1, n_frames, kout), lambda b: (b, 0, 0)),
        scratch_shapes=[
            pltpu.VMEM((yrows, hop), jnp.bfloat16),
            pltpu.VMEM((zrows, nv * hop), jnp.bfloat16),
        ],
        compiler_params=pltpu.CompilerParams(
            dimension_semantics=("parallel",)),
    )(x4, c_d1, c_d2)


def kernel(x):
    cst = _cqt_constants()
    n_bins, fft_len = cst["n_bins"], cst["fft_len"]
    s0 = cst["s0"]

    x = jnp.asarray(x, jnp.float32)
    lead, t_len = x.shape[:-1], x.shape[-1]
    x2 = x.reshape(-1, t_len)
    batch = x2.shape[0]
    n_frames = (t_len - 1) // _HOP + 1

    sig_rows = -(-t_len // _HOP)
    if t_len % _HOP:
        x2 = jnp.pad(x2, ((0, 0), (0, sig_rows * _HOP - t_len)))
    x4 = x2.reshape(batch, sig_rows, _HOP)

    lead_zero = fft_len // 2 // _HOP - s0
    out = _cqt_pallas(x4, cst["c_d1"], cst["c_d2"], n_frames=n_frames,
                      lead_zero=lead_zero, sig_rows=sig_rows, cst=cst,
                      kout=2 * n_bins)
    return out.reshape(*lead, n_frames, n_bins, 2)
